# Initial kernel scaffold; baseline (speedup 1.0000x reference)
import jax
import jax.numpy as jnp
from jax.experimental import pallas as pl


def kernel(patch, pos_emb_table):
    raise NotImplementedError("write your pallas kernel here")


# speedup vs baseline: 1.0130x; 1.0130x over previous
import jax
import jax.numpy as jnp
from jax.experimental import pallas as pl


def _add_body(patch_ref, pos_ref, out_ref):
    out_ref[...] = patch_ref[...] + pos_ref[...]


def kernel(patch, pos_emb_table):
    B, P, D = patch.shape
    grid = (B,)
    return pl.pallas_call(
        _add_body,
        grid=grid,
        in_specs=[
            pl.BlockSpec((1, P, D), lambda b: (b, 0, 0)),
            pl.BlockSpec((P, D), lambda b: (0, 0)),
        ],
        out_specs=pl.BlockSpec((1, P, D), lambda b: (b, 0, 0)),
        out_shape=jax.ShapeDtypeStruct((B, P, D), patch.dtype),
    )(patch, pos_emb_table)
